# baseline (speedup 1.0000x reference)
import functools

import jax
import jax.numpy as jnp
from jax import lax
from jax.experimental import pallas as pl
from jax.experimental.pallas import tpu as pltpu
from jax.experimental.pallas import tpu_sc as plsc


def _pos_encoding(length, hidden_size):
    pos = jnp.arange(length, dtype=jnp.float32)[:, None]
    dims = jnp.arange(hidden_size, dtype=jnp.float32)[None, :]
    angle_rates = jnp.power(10000.0, -2.0 * jnp.floor(dims / 2.0) / float(hidden_size))
    angles = pos * angle_rates
    even = (jnp.arange(hidden_size)[None, :] % 2) == 0
    return jnp.where(even, jnp.sin(angles), jnp.cos(angles))


@functools.lru_cache(maxsize=None)
def _make_sc_gather(n_rows, hidden, chunk):
    info = plsc.get_sparse_core_info()
    nc, ns = info.num_cores, info.num_subcores
    nw = nc * ns
    per_w = n_rows // nw
    assert n_rows % nw == 0 and per_w % chunk == 0
    n_ch = per_w // chunk

    mesh = plsc.VectorSubcoreMesh(core_axis_name="c", subcore_axis_name="s")

    @functools.partial(
        pl.kernel,
        mesh=mesh,
        compiler_params=pltpu.CompilerParams(use_tc_tiling_on_sc=False),
        out_type=jax.ShapeDtypeStruct((n_rows, hidden), jnp.float32),
        scratch_types=[
            pltpu.VMEM((chunk,), jnp.int32),
            pltpu.VMEM((chunk, hidden), jnp.float32),
            pltpu.SemaphoreType.DMA,
        ],
    )
    def sc_kernel(idx_hbm, table, out, idx_v, rows_v, sem):
        wid = lax.axis_index("s") * nc + lax.axis_index("c")
        base = wid * per_w

        def step(i, carry):
            off = base + i * chunk
            pltpu.sync_copy(idx_hbm.at[pl.ds(off, chunk)], idx_v)
            pltpu.async_copy(table.at[idx_v], rows_v, sem).wait()
            pltpu.sync_copy(rows_v, out.at[pl.ds(off, chunk)])
            return carry

        lax.fori_loop(0, n_ch, step, 0)

    return sc_kernel


def _tc_b_body(g, lq, a_ref, eq_ref, peq_ref, b_ref):
    peq = peq_ref[...]
    for i in range(g):
        b_ref[i, :, :] = peq + jnp.dot(
            a_ref[i, :, :], eq_ref[pl.ds(i * lq, lq), :],
            preferred_element_type=jnp.float32,
        )


def _tc_d_body(g, lv, ev_ref, pev_ref, d_ref):
    pev = pev_ref[...]
    for i in range(g):
        d_ref[i, :, :] = pev + ev_ref[pl.ds(i * lv, lv), :]


@functools.lru_cache(maxsize=None)
def _make_tc_b(batch, lq, hidden, g):
    assert batch % g == 0
    return pl.pallas_call(
        functools.partial(_tc_b_body, g, lq),
        grid=(batch // g,),
        in_specs=[
            pl.BlockSpec((g, lq, lq), lambda i: (i, 0, 0)),
            pl.BlockSpec((g * lq, hidden), lambda i: (i, 0)),
            pl.BlockSpec((lq, hidden), lambda i: (0, 0)),
        ],
        out_specs=pl.BlockSpec((g, lq, hidden), lambda i: (i, 0, 0)),
        out_shape=jax.ShapeDtypeStruct((batch, lq, hidden), jnp.float32),
    )


@functools.lru_cache(maxsize=None)
def _make_tc_d(batch, lv, hidden, g):
    assert batch % g == 0
    return pl.pallas_call(
        functools.partial(_tc_d_body, g, lv),
        grid=(batch // g,),
        in_specs=[
            pl.BlockSpec((g * lv, hidden), lambda i: (i, 0)),
            pl.BlockSpec((lv, hidden), lambda i: (0, 0)),
        ],
        out_specs=pl.BlockSpec((g, lv, hidden), lambda i: (i, 0, 0)),
        out_shape=jax.ShapeDtypeStruct((batch, lv, hidden), jnp.float32),
    )


def kernel(queries, values, queries_mask, values_mask, ids, permutation,
           absolute_positions, relative_positions, pointer_labels,
           logits_labels, partial_pos, pointer_probs, log_probs,
           object_detections, object_features, object_boxes,
           src_table, tgt_table):
    batch, lq = queries.shape
    lv = values.shape[1]
    hidden = tgt_table.shape[1]

    q_flat = queries.reshape(-1).astype(jnp.int32)
    v_flat = values.reshape(-1).astype(jnp.int32)

    sc_q = _make_sc_gather(batch * lq, hidden, 1280)
    sc_v = _make_sc_gather(batch * lv, hidden, 1280)
    eq = sc_q(q_flat, tgt_table)
    ev = sc_v(v_flat, src_table)

    peq = _pos_encoding(lq, hidden)
    pev = _pos_encoding(lv, hidden)

    b = _make_tc_b(batch, lq, hidden, 64)(absolute_positions, eq, peq)
    d = _make_tc_d(batch, lv, hidden, 64)(ev, pev)

    return (b, d, queries_mask, values_mask, ids, permutation,
            absolute_positions, relative_positions, pointer_labels,
            logits_labels, partial_pos, pointer_probs, log_probs,
            object_detections, object_features, object_boxes)


# speedup vs baseline: 1.2132x; 1.2132x over previous
import functools

import jax
import jax.numpy as jnp
from jax import lax
from jax.experimental import pallas as pl
from jax.experimental.pallas import tpu as pltpu
from jax.experimental.pallas import tpu_sc as plsc


def _pos_encoding(length, hidden_size):
    pos = jnp.arange(length, dtype=jnp.float32)[:, None]
    dims = jnp.arange(hidden_size, dtype=jnp.float32)[None, :]
    angle_rates = jnp.power(10000.0, -2.0 * jnp.floor(dims / 2.0) / float(hidden_size))
    angles = pos * angle_rates
    even = (jnp.arange(hidden_size)[None, :] % 2) == 0
    return jnp.where(even, jnp.sin(angles), jnp.cos(angles))


@functools.lru_cache(maxsize=None)
def _make_sc_gather(n_rows, hidden, chunk):
    info = plsc.get_sparse_core_info()
    nc, ns = info.num_cores, info.num_subcores
    nw = nc * ns
    per_w = n_rows // nw
    assert n_rows % nw == 0 and per_w % chunk == 0
    n_ch = per_w // chunk

    mesh = plsc.VectorSubcoreMesh(core_axis_name="c", subcore_axis_name="s")

    @functools.partial(
        pl.kernel,
        mesh=mesh,
        compiler_params=pltpu.CompilerParams(use_tc_tiling_on_sc=False),
        out_type=jax.ShapeDtypeStruct((n_rows, 2 * hidden), jnp.float32),
        scratch_types=[
            pltpu.VMEM((chunk,), jnp.int32),
            pltpu.VMEM((chunk, hidden), jnp.float32),
            pltpu.SemaphoreType.DMA,
        ],
    )
    def sc_kernel(idx_hbm, table, out, idx_v, rows_v, sem):
        wid = lax.axis_index("s") * nc + lax.axis_index("c")
        base = wid * per_w

        def step(i, carry):
            off = base + i * chunk
            pltpu.sync_copy(idx_hbm.at[pl.ds(off, chunk)], idx_v)
            pltpu.async_copy(table.at[idx_v], rows_v, sem).wait()
            pltpu.sync_copy(rows_v, out.at[pl.ds(off, chunk), pl.ds(0, hidden)])
            return carry

        lax.fori_loop(0, n_ch, step, 0)

    return sc_kernel


def _tc_b_body(g, lq, hidden, a_ref, eq_ref, peq_ref, b_ref):
    peq = peq_ref[...]
    for i in range(g):
        b_ref[i, :, :] = peq + jnp.dot(
            a_ref[i, :, :], eq_ref[pl.ds(i * lq, lq), pl.ds(0, hidden)],
            preferred_element_type=jnp.float32,
        )


def _tc_d_body(g, lv, hidden, ev_ref, pev_ref, d_ref):
    pev = pev_ref[...]
    for i in range(g):
        d_ref[i, :, :] = pev + ev_ref[pl.ds(i * lv, lv), pl.ds(0, hidden)]


@functools.lru_cache(maxsize=None)
def _make_tc_b(batch, lq, hidden, g):
    assert batch % g == 0
    return pl.pallas_call(
        functools.partial(_tc_b_body, g, lq, hidden),
        grid=(batch // g,),
        in_specs=[
            pl.BlockSpec((g, lq, lq), lambda i: (i, 0, 0)),
            pl.BlockSpec((g * lq, 2 * hidden), lambda i: (i, 0)),
            pl.BlockSpec((lq, hidden), lambda i: (0, 0)),
        ],
        out_specs=pl.BlockSpec((g, lq, hidden), lambda i: (i, 0, 0)),
        out_shape=jax.ShapeDtypeStruct((batch, lq, hidden), jnp.float32),
    )


@functools.lru_cache(maxsize=None)
def _make_tc_d(batch, lv, hidden, g):
    assert batch % g == 0
    return pl.pallas_call(
        functools.partial(_tc_d_body, g, lv, hidden),
        grid=(batch // g,),
        in_specs=[
            pl.BlockSpec((g * lv, 2 * hidden), lambda i: (i, 0)),
            pl.BlockSpec((lv, hidden), lambda i: (0, 0)),
        ],
        out_specs=pl.BlockSpec((g, lv, hidden), lambda i: (i, 0, 0)),
        out_shape=jax.ShapeDtypeStruct((batch, lv, hidden), jnp.float32),
    )


def kernel(queries, values, queries_mask, values_mask, ids, permutation,
           absolute_positions, relative_positions, pointer_labels,
           logits_labels, partial_pos, pointer_probs, log_probs,
           object_detections, object_features, object_boxes,
           src_table, tgt_table):
    batch, lq = queries.shape
    lv = values.shape[1]
    hidden = tgt_table.shape[1]

    q_flat = queries.reshape(-1).astype(jnp.int32)
    v_flat = values.reshape(-1).astype(jnp.int32)

    sc_q = _make_sc_gather(batch * lq, hidden, 1280)
    sc_v = _make_sc_gather(batch * lv, hidden, 1280)
    eq = sc_q(q_flat, tgt_table)
    ev = sc_v(v_flat, src_table)

    peq = _pos_encoding(lq, hidden)
    pev = _pos_encoding(lv, hidden)

    b = _make_tc_b(batch, lq, hidden, 64)(absolute_positions, eq, peq)
    d = _make_tc_d(batch, lv, hidden, 64)(ev, pev)

    return (b, d, queries_mask, values_mask, ids, permutation,
            absolute_positions, relative_positions, pointer_labels,
            logits_labels, partial_pos, pointer_probs, log_probs,
            object_detections, object_features, object_boxes)


# speedup vs baseline: 1.2403x; 1.0223x over previous
import functools

import jax
import jax.numpy as jnp
from jax import lax
from jax.experimental import pallas as pl
from jax.experimental.pallas import tpu as pltpu
from jax.experimental.pallas import tpu_sc as plsc


def _pos_encoding(length, hidden_size):
    pos = jnp.arange(length, dtype=jnp.float32)[:, None]
    dims = jnp.arange(hidden_size, dtype=jnp.float32)[None, :]
    angle_rates = jnp.power(10000.0, -2.0 * jnp.floor(dims / 2.0) / float(hidden_size))
    angles = pos * angle_rates
    even = (jnp.arange(hidden_size)[None, :] % 2) == 0
    return jnp.where(even, jnp.sin(angles), jnp.cos(angles))


@functools.lru_cache(maxsize=None)
def _make_sc_gather(n_rows, hidden, chunk):
    info = plsc.get_sparse_core_info()
    nc, ns = info.num_cores, info.num_subcores
    nw = nc * ns
    per_w = n_rows // nw
    assert n_rows % nw == 0 and per_w % chunk == 0
    n_ch = per_w // chunk

    mesh = plsc.VectorSubcoreMesh(core_axis_name="c", subcore_axis_name="s")

    @functools.partial(
        pl.kernel,
        mesh=mesh,
        compiler_params=pltpu.CompilerParams(use_tc_tiling_on_sc=False),
        out_type=jax.ShapeDtypeStruct((n_rows, 2 * hidden), jnp.float32),
        scratch_types=[
            pltpu.VMEM((chunk,), jnp.int32),
            pltpu.VMEM((chunk, hidden), jnp.float32),
            pltpu.SemaphoreType.DMA,
        ],
    )
    def sc_kernel(idx_hbm, table, out, idx_v, rows_v, sem):
        wid = lax.axis_index("s") * nc + lax.axis_index("c")
        base = wid * per_w

        def step(i, carry):
            off = base + i * chunk
            pltpu.sync_copy(idx_hbm.at[pl.ds(off, chunk)], idx_v)
            pltpu.async_copy(table.at[idx_v], rows_v, sem).wait()
            pltpu.sync_copy(rows_v, out.at[pl.ds(off, chunk), pl.ds(0, hidden)])
            return carry

        lax.fori_loop(0, n_ch, step, 0)

    return sc_kernel


def _tc_b_body(g, lq, hidden, a_ref, eq_ref, peq_ref, b_ref):
    peq = peq_ref[...]
    for i in range(g):
        b_ref[i, :, :] = peq + jnp.dot(
            a_ref[i, :, :], eq_ref[pl.ds(i * lq, lq), pl.ds(0, hidden)],
            preferred_element_type=jnp.float32,
        )


def _tc_d_body(g, lv, hidden, ev_ref, pev_ref, d_ref):
    pev = pev_ref[...]
    for i in range(g):
        d_ref[i, :, :] = pev + ev_ref[pl.ds(i * lv, lv), pl.ds(0, hidden)]


@functools.lru_cache(maxsize=None)
def _make_tc_b(batch, lq, hidden, g):
    assert batch % g == 0
    return pl.pallas_call(
        functools.partial(_tc_b_body, g, lq, hidden),
        grid=(batch // g,),
        in_specs=[
            pl.BlockSpec((g, lq, lq), lambda i: (i, 0, 0)),
            pl.BlockSpec((g * lq, 2 * hidden), lambda i: (i, 0)),
            pl.BlockSpec((lq, hidden), lambda i: (0, 0)),
        ],
        out_specs=pl.BlockSpec((g, lq, hidden), lambda i: (i, 0, 0)),
        out_shape=jax.ShapeDtypeStruct((batch, lq, hidden), jnp.float32),
    )


@functools.lru_cache(maxsize=None)
def _make_tc_d(batch, lv, hidden, g):
    assert batch % g == 0
    return pl.pallas_call(
        functools.partial(_tc_d_body, g, lv, hidden),
        grid=(batch // g,),
        in_specs=[
            pl.BlockSpec((g * lv, 2 * hidden), lambda i: (i, 0)),
            pl.BlockSpec((lv, hidden), lambda i: (0, 0)),
        ],
        out_specs=pl.BlockSpec((g, lv, hidden), lambda i: (i, 0, 0)),
        out_shape=jax.ShapeDtypeStruct((batch, lv, hidden), jnp.float32),
    )


def kernel(queries, values, queries_mask, values_mask, ids, permutation,
           absolute_positions, relative_positions, pointer_labels,
           logits_labels, partial_pos, pointer_probs, log_probs,
           object_detections, object_features, object_boxes,
           src_table, tgt_table):
    batch, lq = queries.shape
    lv = values.shape[1]
    hidden = tgt_table.shape[1]

    q_flat = queries.reshape(-1).astype(jnp.int32)
    v_flat = values.reshape(-1).astype(jnp.int32)

    sc_q = _make_sc_gather(batch * lq, hidden, 1280)
    sc_v = _make_sc_gather(batch * lv, hidden, 1280)
    eq = sc_q(q_flat, tgt_table)
    ev = sc_v(v_flat, src_table)

    peq = _pos_encoding(lq, hidden)
    pev = _pos_encoding(lv, hidden)

    b = _make_tc_b(batch, lq, hidden, 128)(absolute_positions, eq, peq)
    d = _make_tc_d(batch, lv, hidden, 128)(ev, pev)

    return (b, d, queries_mask, values_mask, ids, permutation,
            absolute_positions, relative_positions, pointer_labels,
            logits_labels, partial_pos, pointer_probs, log_probs,
            object_detections, object_features, object_boxes)


# speedup vs baseline: 1.2448x; 1.0037x over previous
import functools

import jax
import jax.numpy as jnp
from jax import lax
from jax.experimental import pallas as pl
from jax.experimental.pallas import tpu as pltpu
from jax.experimental.pallas import tpu_sc as plsc


def _pos_encoding(length, hidden_size):
    pos = jnp.arange(length, dtype=jnp.float32)[:, None]
    dims = jnp.arange(hidden_size, dtype=jnp.float32)[None, :]
    angle_rates = jnp.power(10000.0, -2.0 * jnp.floor(dims / 2.0) / float(hidden_size))
    angles = pos * angle_rates
    even = (jnp.arange(hidden_size)[None, :] % 2) == 0
    return jnp.where(even, jnp.sin(angles), jnp.cos(angles))


@functools.lru_cache(maxsize=None)
def _make_sc_gather(n_rows, hidden, chunk):
    info = plsc.get_sparse_core_info()
    nc, ns = info.num_cores, info.num_subcores
    nw = nc * ns
    per_w = n_rows // nw
    assert n_rows % nw == 0 and per_w % chunk == 0
    n_ch = per_w // chunk

    mesh = plsc.VectorSubcoreMesh(core_axis_name="c", subcore_axis_name="s")

    @functools.partial(
        pl.kernel,
        mesh=mesh,
        compiler_params=pltpu.CompilerParams(use_tc_tiling_on_sc=False),
        out_type=jax.ShapeDtypeStruct((n_rows, 2 * hidden), jnp.float32),
        scratch_types=[
            pltpu.VMEM((chunk,), jnp.int32),
            pltpu.VMEM((chunk, hidden), jnp.float32),
            pltpu.SemaphoreType.DMA,
        ],
    )
    def sc_kernel(idx_hbm, table, out, idx_v, rows_v, sem):
        wid = lax.axis_index("s") * nc + lax.axis_index("c")
        base = wid * per_w

        def step(i, carry):
            off = base + i * chunk
            pltpu.sync_copy(idx_hbm.at[pl.ds(off, chunk)], idx_v)
            pltpu.async_copy(table.at[idx_v], rows_v, sem).wait()
            pltpu.sync_copy(rows_v, out.at[pl.ds(off, chunk), pl.ds(0, hidden)])
            return carry

        lax.fori_loop(0, n_ch, step, 0)

    return sc_kernel


def _tc_b_body(g, lq, hidden, a_ref, eq_ref, peq_ref, b_ref):
    peq = peq_ref[...]
    for i in range(g):
        b_ref[i, :, :] = peq + jnp.dot(
            a_ref[i, :, :], eq_ref[pl.ds(i * lq, lq), pl.ds(0, hidden)],
            preferred_element_type=jnp.float32,
        )


def _tc_d_body(g, lv, hidden, ev_ref, pev_ref, d_ref):
    pev = pev_ref[...]
    for i in range(g):
        d_ref[i, :, :] = pev + ev_ref[pl.ds(i * lv, lv), pl.ds(0, hidden)]


@functools.lru_cache(maxsize=None)
def _make_tc_b(batch, lq, hidden, g):
    assert batch % g == 0
    return pl.pallas_call(
        functools.partial(_tc_b_body, g, lq, hidden),
        grid=(batch // g,),
        in_specs=[
            pl.BlockSpec((g, lq, lq), lambda i: (i, 0, 0)),
            pl.BlockSpec((g * lq, 2 * hidden), lambda i: (i, 0)),
            pl.BlockSpec((lq, hidden), lambda i: (0, 0)),
        ],
        out_specs=pl.BlockSpec((g, lq, hidden), lambda i: (i, 0, 0)),
        out_shape=jax.ShapeDtypeStruct((batch, lq, hidden), jnp.float32),
    )


@functools.lru_cache(maxsize=None)
def _make_tc_d(batch, lv, hidden, g):
    assert batch % g == 0
    return pl.pallas_call(
        functools.partial(_tc_d_body, g, lv, hidden),
        grid=(batch // g,),
        in_specs=[
            pl.BlockSpec((g * lv, 2 * hidden), lambda i: (i, 0)),
            pl.BlockSpec((lv, hidden), lambda i: (0, 0)),
        ],
        out_specs=pl.BlockSpec((g, lv, hidden), lambda i: (i, 0, 0)),
        out_shape=jax.ShapeDtypeStruct((batch, lv, hidden), jnp.float32),
    )


def kernel(queries, values, queries_mask, values_mask, ids, permutation,
           absolute_positions, relative_positions, pointer_labels,
           logits_labels, partial_pos, pointer_probs, log_probs,
           object_detections, object_features, object_boxes,
           src_table, tgt_table):
    batch, lq = queries.shape
    lv = values.shape[1]
    hidden = tgt_table.shape[1]

    q_flat = queries.reshape(-1).astype(jnp.int32)
    v_flat = values.reshape(-1).astype(jnp.int32)

    sc_q = _make_sc_gather(batch * lq, hidden, 1280)
    sc_v = _make_sc_gather(batch * lv, hidden, 1280)
    eq = sc_q(q_flat, tgt_table)
    ev = sc_v(v_flat, src_table)

    peq = _pos_encoding(lq, hidden)
    pev = _pos_encoding(lv, hidden)

    b = _make_tc_b(batch, lq, hidden, 256)(absolute_positions, eq, peq)
    d = _make_tc_d(batch, lv, hidden, 256)(ev, pev)

    return (b, d, queries_mask, values_mask, ids, permutation,
            absolute_positions, relative_positions, pointer_labels,
            logits_labels, partial_pos, pointer_probs, log_probs,
            object_detections, object_features, object_boxes)


# speedup vs baseline: 1.2455x; 1.0006x over previous
import functools

import jax
import jax.numpy as jnp
from jax import lax
from jax.experimental import pallas as pl
from jax.experimental.pallas import tpu as pltpu
from jax.experimental.pallas import tpu_sc as plsc


def _pos_encoding(length, hidden_size):
    pos = jnp.arange(length, dtype=jnp.float32)[:, None]
    dims = jnp.arange(hidden_size, dtype=jnp.float32)[None, :]
    angle_rates = jnp.power(10000.0, -2.0 * jnp.floor(dims / 2.0) / float(hidden_size))
    angles = pos * angle_rates
    even = (jnp.arange(hidden_size)[None, :] % 2) == 0
    return jnp.where(even, jnp.sin(angles), jnp.cos(angles))


@functools.lru_cache(maxsize=None)
def _make_sc_gather(n_rows, hidden, chunk):
    info = plsc.get_sparse_core_info()
    nc, ns = info.num_cores, info.num_subcores
    nw = nc * ns
    per_w = n_rows // nw
    assert n_rows % nw == 0 and per_w % chunk == 0
    n_ch = per_w // chunk

    mesh = plsc.VectorSubcoreMesh(core_axis_name="c", subcore_axis_name="s")

    @functools.partial(
        pl.kernel,
        mesh=mesh,
        compiler_params=pltpu.CompilerParams(use_tc_tiling_on_sc=False),
        out_type=jax.ShapeDtypeStruct((n_rows, 2 * hidden), jnp.float32),
        scratch_types=[
            pltpu.VMEM((chunk,), jnp.int32),
            pltpu.VMEM((chunk, hidden), jnp.float32),
            pltpu.SemaphoreType.DMA,
        ],
    )
    def sc_kernel(idx_hbm, table, out, idx_v, rows_v, sem):
        wid = lax.axis_index("s") * nc + lax.axis_index("c")
        base = wid * per_w

        def step(i, carry):
            off = base + i * chunk
            pltpu.sync_copy(idx_hbm.at[pl.ds(off, chunk)], idx_v)
            pltpu.async_copy(table.at[idx_v], rows_v, sem).wait()
            pltpu.sync_copy(rows_v, out.at[pl.ds(off, chunk), pl.ds(0, hidden)])
            return carry

        lax.fori_loop(0, n_ch, step, 0)

    return sc_kernel


def _tc_b_body(g, lq, hidden, a_ref, eq_ref, ev_ref, peq_ref, pev_ref, b_ref, d_ref):
    peq = peq_ref[...]
    pev = pev_ref[...]
    for i in range(g):
        b_ref[i, :, :] = peq + jnp.dot(
            a_ref[i, :, :], eq_ref[pl.ds(i * lq, lq), pl.ds(0, hidden)],
            preferred_element_type=jnp.float32,
        )
        d_ref[i, :, :] = pev + ev_ref[pl.ds(i * lq, lq), pl.ds(0, hidden)]


def _tc_d_body(g, lv, hidden, ev_ref, pev_ref, d_ref):
    pev = pev_ref[...]
    for i in range(g):
        d_ref[i, :, :] = pev + ev_ref[pl.ds(i * lv, lv), pl.ds(0, hidden)]


@functools.lru_cache(maxsize=None)
def _make_tc_b(batch, lq, hidden, g):
    assert batch % g == 0
    return pl.pallas_call(
        functools.partial(_tc_b_body, g, lq, hidden),
        grid=(batch // g,),
        in_specs=[
            pl.BlockSpec((g, lq, lq), lambda i: (i, 0, 0)),
            pl.BlockSpec((g * lq, 2 * hidden), lambda i: (i, 0)),
            pl.BlockSpec((g * lq, 2 * hidden), lambda i: (i, 0)),
            pl.BlockSpec((lq, hidden), lambda i: (0, 0)),
            pl.BlockSpec((lq, hidden), lambda i: (0, 0)),
        ],
        out_specs=[
            pl.BlockSpec((g, lq, hidden), lambda i: (i, 0, 0)),
            pl.BlockSpec((g, lq, hidden), lambda i: (i, 0, 0)),
        ],
        out_shape=[
            jax.ShapeDtypeStruct((batch, lq, hidden), jnp.float32),
            jax.ShapeDtypeStruct((batch, lq, hidden), jnp.float32),
        ],
    )


@functools.lru_cache(maxsize=None)
def _make_tc_d(batch, lv, hidden, g):
    assert batch % g == 0
    return pl.pallas_call(
        functools.partial(_tc_d_body, g, lv, hidden),
        grid=(batch // g,),
        in_specs=[
            pl.BlockSpec((g * lv, 2 * hidden), lambda i: (i, 0)),
            pl.BlockSpec((lv, hidden), lambda i: (0, 0)),
        ],
        out_specs=pl.BlockSpec((g, lv, hidden), lambda i: (i, 0, 0)),
        out_shape=jax.ShapeDtypeStruct((batch, lv, hidden), jnp.float32),
    )


def kernel(queries, values, queries_mask, values_mask, ids, permutation,
           absolute_positions, relative_positions, pointer_labels,
           logits_labels, partial_pos, pointer_probs, log_probs,
           object_detections, object_features, object_boxes,
           src_table, tgt_table):
    batch, lq = queries.shape
    lv = values.shape[1]
    hidden = tgt_table.shape[1]

    q_flat = queries.reshape(-1).astype(jnp.int32)
    v_flat = values.reshape(-1).astype(jnp.int32)

    sc_q = _make_sc_gather(batch * lq, hidden, 1280)
    sc_v = _make_sc_gather(batch * lv, hidden, 1280)
    eq = sc_q(q_flat, tgt_table)
    ev = sc_v(v_flat, src_table)

    peq = _pos_encoding(lq, hidden)
    pev = _pos_encoding(lv, hidden)

    b, d = _make_tc_b(batch, lq, hidden, 128)(absolute_positions, eq, ev, peq, pev)

    return (b, d, queries_mask, values_mask, ids, permutation,
            absolute_positions, relative_positions, pointer_labels,
            logits_labels, partial_pos, pointer_probs, log_probs,
            object_detections, object_features, object_boxes)


# speedup vs baseline: 1.3202x; 1.0600x over previous
import functools

import jax
import jax.numpy as jnp
from jax import lax
from jax.experimental import pallas as pl
from jax.experimental.pallas import tpu as pltpu
from jax.experimental.pallas import tpu_sc as plsc


def _pos_encoding(length, hidden_size):
    pos = jnp.arange(length, dtype=jnp.float32)[:, None]
    dims = jnp.arange(hidden_size, dtype=jnp.float32)[None, :]
    angle_rates = jnp.power(10000.0, -2.0 * jnp.floor(dims / 2.0) / float(hidden_size))
    angles = pos * angle_rates
    even = (jnp.arange(hidden_size)[None, :] % 2) == 0
    return jnp.where(even, jnp.sin(angles), jnp.cos(angles))


@functools.lru_cache(maxsize=None)
def _make_sc_gather_pack(n_rows, hidden, chunk):
    info = plsc.get_sparse_core_info()
    nc, ns = info.num_cores, info.num_subcores
    nw = nc * ns
    per_w = n_rows // nw
    assert n_rows % nw == 0 and per_w % chunk == 0 and chunk % 100 == 0
    n_ch = per_w // chunk
    n_vec = hidden // 16
    n_grp = chunk // 100

    mesh = plsc.VectorSubcoreMesh(core_axis_name="c", subcore_axis_name="s")

    @functools.partial(
        pl.kernel,
        mesh=mesh,
        compiler_params=pltpu.CompilerParams(use_tc_tiling_on_sc=False, needs_layout_passes=False),
        out_type=jax.ShapeDtypeStruct((n_rows // 4, 2 * hidden), jnp.float32),
        scratch_types=[
            pltpu.VMEM((chunk,), jnp.int32),
            pltpu.VMEM((chunk, hidden), jnp.float32),
            pltpu.VMEM((chunk // 4, 2 * hidden), jnp.float32),
            pltpu.SemaphoreType.DMA,
        ],
    )
    def sc_kernel(idx_hbm, table, out, idx_v, rows_v, pack_v, sem):
        wid = lax.axis_index("s") * nc + lax.axis_index("c")
        base = wid * per_w

        def pack_group(g2, carry):
            rbase = g2 * 100
            pbase = g2 * 25

            def pack_p(rel, carry2):
                p = pbase + rel
                for half in range(2):
                    ra = rbase + 25 * half + rel
                    rb = ra + 50
                    for h in range(n_vec):
                        va = rows_v[ra, pl.ds(16 * h, 16)]
                        vb = rows_v[rb, pl.ds(16 * h, 16)]
                        pk = plsc.pack(va, vb, format=plsc.PackFormat.INTERLEAVED)
                        w = plsc.bitcast(pk, jnp.float32)
                        pack_v[p, pl.ds(half * hidden + 16 * h, 16)] = w
                return carry2

            lax.fori_loop(0, 25, pack_p, 0)
            return carry

        def step(i, carry):
            off = base + i * chunk
            pltpu.sync_copy(idx_hbm.at[pl.ds(off, chunk)], idx_v)
            pltpu.async_copy(table.at[idx_v], rows_v, sem).wait()
            lax.fori_loop(0, n_grp, pack_group, 0)
            pltpu.sync_copy(pack_v, out.at[pl.ds(off // 4, chunk // 4), :])
            return carry

        lax.fori_loop(0, n_ch, step, 0)

    return sc_kernel


def _tc_b_body(g, lq, hidden, a_ref, eq_ref, peq_ref, b_ref):
    peq = peq_ref[...]
    wi = pltpu.bitcast(eq_ref[...], jnp.uint32)
    lo = pltpu.bitcast(wi << 16, jnp.float32)
    hi = pltpu.bitcast(wi & jnp.uint32(0xFFFF0000), jnp.float32)
    half = lq // 2
    for k in range(g // 2):
        l = lo[k * half:(k + 1) * half, :]
        h_ = hi[k * half:(k + 1) * half, :]
        a0 = a_ref[2 * k, :, :]
        a1 = a_ref[2 * k + 1, :, :]
        b_ref[2 * k, :, :] = peq + jnp.dot(
            a0[:, :half], l[:, :hidden],
            preferred_element_type=jnp.float32,
        ) + jnp.dot(
            a0[:, half:], l[:, hidden:],
            preferred_element_type=jnp.float32,
        )
        b_ref[2 * k + 1, :, :] = peq + jnp.dot(
            a1[:, :half], h_[:, :hidden],
            preferred_element_type=jnp.float32,
        ) + jnp.dot(
            a1[:, half:], h_[:, hidden:],
            preferred_element_type=jnp.float32,
        )


def _tc_d_body(g, lv, hidden, ev_ref, pev_ref, d_ref):
    pev = pev_ref[...]
    wi = pltpu.bitcast(ev_ref[...], jnp.uint32)
    lo = pltpu.bitcast(wi << 16, jnp.float32)
    hi = pltpu.bitcast(wi & jnp.uint32(0xFFFF0000), jnp.float32)
    half = lv // 2
    for k in range(g // 2):
        l = lo[k * half:(k + 1) * half, :]
        h_ = hi[k * half:(k + 1) * half, :]
        d_ref[2 * k, :half, :] = pev[:half, :] + l[:, :hidden]
        d_ref[2 * k, half:, :] = pev[half:, :] + l[:, hidden:]
        d_ref[2 * k + 1, :half, :] = pev[:half, :] + h_[:, :hidden]
        d_ref[2 * k + 1, half:, :] = pev[half:, :] + h_[:, hidden:]


@functools.lru_cache(maxsize=None)
def _make_tc_b(batch, lq, hidden, g):
    assert batch % g == 0 and g % 2 == 0 and lq % 2 == 0
    return pl.pallas_call(
        functools.partial(_tc_b_body, g, lq, hidden),
        grid=(batch // g,),
        in_specs=[
            pl.BlockSpec((g, lq, lq), lambda i: (i, 0, 0)),
            pl.BlockSpec((g * lq // 4, 2 * hidden), lambda i: (i, 0)),
            pl.BlockSpec((lq, hidden), lambda i: (0, 0)),
        ],
        out_specs=pl.BlockSpec((g, lq, hidden), lambda i: (i, 0, 0)),
        out_shape=jax.ShapeDtypeStruct((batch, lq, hidden), jnp.float32),
    )


@functools.lru_cache(maxsize=None)
def _make_tc_d(batch, lv, hidden, g):
    assert batch % g == 0 and g % 2 == 0 and lv % 2 == 0
    return pl.pallas_call(
        functools.partial(_tc_d_body, g, lv, hidden),
        grid=(batch // g,),
        in_specs=[
            pl.BlockSpec((g * lv // 4, 2 * hidden), lambda i: (i, 0)),
            pl.BlockSpec((lv, hidden), lambda i: (0, 0)),
        ],
        out_specs=pl.BlockSpec((g, lv, hidden), lambda i: (i, 0, 0)),
        out_shape=jax.ShapeDtypeStruct((batch, lv, hidden), jnp.float32),
    )


def kernel(queries, values, queries_mask, values_mask, ids, permutation,
           absolute_positions, relative_positions, pointer_labels,
           logits_labels, partial_pos, pointer_probs, log_probs,
           object_detections, object_features, object_boxes,
           src_table, tgt_table):
    batch, lq = queries.shape
    lv = values.shape[1]
    hidden = tgt_table.shape[1]

    q_flat = queries.reshape(-1).astype(jnp.int32)
    v_flat = values.reshape(-1).astype(jnp.int32)

    sc_q = _make_sc_gather_pack(batch * lq, hidden, 800)
    sc_v = _make_sc_gather_pack(batch * lv, hidden, 800)
    eq = sc_q(q_flat, tgt_table)
    ev = sc_v(v_flat, src_table)

    peq = _pos_encoding(lq, hidden)
    pev = _pos_encoding(lv, hidden)

    b = _make_tc_b(batch, lq, hidden, 128)(absolute_positions, eq, peq)
    d = _make_tc_d(batch, lv, hidden, 128)(ev, pev)

    return (b, d, queries_mask, values_mask, ids, permutation,
            absolute_positions, relative_positions, pointer_labels,
            logits_labels, partial_pos, pointer_probs, log_probs,
            object_detections, object_features, object_boxes)


# speedup vs baseline: 1.3394x; 1.0146x over previous
import functools

import jax
import jax.numpy as jnp
from jax import lax
from jax.experimental import pallas as pl
from jax.experimental.pallas import tpu as pltpu
from jax.experimental.pallas import tpu_sc as plsc


def _pos_encoding(length, hidden_size):
    pos = jnp.arange(length, dtype=jnp.float32)[:, None]
    dims = jnp.arange(hidden_size, dtype=jnp.float32)[None, :]
    angle_rates = jnp.power(10000.0, -2.0 * jnp.floor(dims / 2.0) / float(hidden_size))
    angles = pos * angle_rates
    even = (jnp.arange(hidden_size)[None, :] % 2) == 0
    return jnp.where(even, jnp.sin(angles), jnp.cos(angles))


@functools.lru_cache(maxsize=None)
def _make_sc_gather_pack(n_rows, hidden, chunk):
    info = plsc.get_sparse_core_info()
    nc, ns = info.num_cores, info.num_subcores
    nw = nc * ns
    per_w = n_rows // nw
    assert n_rows % nw == 0 and per_w % chunk == 0 and chunk % 100 == 0
    n_ch = per_w // chunk
    n_vec = hidden // 16
    n_grp = chunk // 100

    mesh = plsc.VectorSubcoreMesh(core_axis_name="c", subcore_axis_name="s")

    @functools.partial(
        pl.kernel,
        mesh=mesh,
        compiler_params=pltpu.CompilerParams(use_tc_tiling_on_sc=False, needs_layout_passes=False),
        out_type=jax.ShapeDtypeStruct((n_rows // 4, 2 * hidden), jnp.float32),
        scratch_types=[
            pltpu.VMEM((chunk,), jnp.int32),
            pltpu.VMEM((chunk, hidden), jnp.float32),
            pltpu.VMEM((chunk // 4, 2 * hidden), jnp.float32),
            pltpu.SemaphoreType.DMA,
        ],
    )
    def sc_kernel(idx_hbm, table, out, idx_v, rows_v, pack_v, sem):
        wid = lax.axis_index("s") * nc + lax.axis_index("c")
        base = wid * per_w

        def pack_group(g2, carry):
            rbase = g2 * 100
            pbase = g2 * 25

            def pack_p(rel, carry2):
                p = pbase + rel
                for half in range(2):
                    ra = rbase + 25 * half + rel
                    rb = ra + 50
                    for h in range(n_vec):
                        va = rows_v[ra, pl.ds(16 * h, 16)]
                        vb = rows_v[rb, pl.ds(16 * h, 16)]
                        pk = plsc.pack(va, vb, format=plsc.PackFormat.INTERLEAVED)
                        w = plsc.bitcast(pk, jnp.float32)
                        pack_v[p, pl.ds(half * hidden + 16 * h, 16)] = w
                return carry2

            lax.fori_loop(0, 25, pack_p, 0)
            return carry

        def step(i, carry):
            off = base + i * chunk
            pltpu.sync_copy(idx_hbm.at[pl.ds(off, chunk)], idx_v)
            pltpu.async_copy(table.at[idx_v], rows_v, sem).wait()
            lax.fori_loop(0, n_grp, pack_group, 0)
            pltpu.sync_copy(pack_v, out.at[pl.ds(off // 4, chunk // 4), :])
            return carry

        lax.fori_loop(0, n_ch, step, 0)

    return sc_kernel


def _tc_b_body(g, lq, hidden, a_ref, eq_ref, peq_ref, b_ref):
    peq = peq_ref[...]
    wi = pltpu.bitcast(eq_ref[...], jnp.uint32)
    lo = pltpu.bitcast(wi << 16, jnp.float32)
    hi = pltpu.bitcast(wi & jnp.uint32(0xFFFF0000), jnp.float32)
    half = lq // 2
    for k in range(g // 2):
        l = lo[k * half:(k + 1) * half, :]
        h_ = hi[k * half:(k + 1) * half, :]
        a0 = a_ref[2 * k, :, :]
        a1 = a_ref[2 * k + 1, :, :]
        b_ref[2 * k, :, :] = peq + jnp.dot(
            a0[:, :half], l[:, :hidden],
            preferred_element_type=jnp.float32,
        ) + jnp.dot(
            a0[:, half:], l[:, hidden:],
            preferred_element_type=jnp.float32,
        )
        b_ref[2 * k + 1, :, :] = peq + jnp.dot(
            a1[:, :half], h_[:, :hidden],
            preferred_element_type=jnp.float32,
        ) + jnp.dot(
            a1[:, half:], h_[:, hidden:],
            preferred_element_type=jnp.float32,
        )


def _tc_d_body(g, lv, hidden, ev_ref, pev_ref, d_ref):
    pev = pev_ref[...]
    wi = pltpu.bitcast(ev_ref[...], jnp.uint32)
    lo = pltpu.bitcast(wi << 16, jnp.float32)
    hi = pltpu.bitcast(wi & jnp.uint32(0xFFFF0000), jnp.float32)
    half = lv // 2
    for k in range(g // 2):
        l = lo[k * half:(k + 1) * half, :]
        h_ = hi[k * half:(k + 1) * half, :]
        d_ref[2 * k, :half, :] = pev[:half, :] + l[:, :hidden]
        d_ref[2 * k, half:, :] = pev[half:, :] + l[:, hidden:]
        d_ref[2 * k + 1, :half, :] = pev[:half, :] + h_[:, :hidden]
        d_ref[2 * k + 1, half:, :] = pev[half:, :] + h_[:, hidden:]


@functools.lru_cache(maxsize=None)
def _make_tc_b(batch, lq, hidden, g):
    assert batch % g == 0 and g % 2 == 0 and lq % 2 == 0
    return pl.pallas_call(
        functools.partial(_tc_b_body, g, lq, hidden),
        grid=(batch // g,),
        in_specs=[
            pl.BlockSpec((g, lq, lq), lambda i: (i, 0, 0)),
            pl.BlockSpec((g * lq // 4, 2 * hidden), lambda i: (i, 0)),
            pl.BlockSpec((lq, hidden), lambda i: (0, 0)),
        ],
        out_specs=pl.BlockSpec((g, lq, hidden), lambda i: (i, 0, 0)),
        out_shape=jax.ShapeDtypeStruct((batch, lq, hidden), jnp.float32),
    )


@functools.lru_cache(maxsize=None)
def _make_tc_d(batch, lv, hidden, g):
    assert batch % g == 0 and g % 2 == 0 and lv % 2 == 0
    return pl.pallas_call(
        functools.partial(_tc_d_body, g, lv, hidden),
        grid=(batch // g,),
        in_specs=[
            pl.BlockSpec((g * lv // 4, 2 * hidden), lambda i: (i, 0)),
            pl.BlockSpec((lv, hidden), lambda i: (0, 0)),
        ],
        out_specs=pl.BlockSpec((g, lv, hidden), lambda i: (i, 0, 0)),
        out_shape=jax.ShapeDtypeStruct((batch, lv, hidden), jnp.float32),
    )


def kernel(queries, values, queries_mask, values_mask, ids, permutation,
           absolute_positions, relative_positions, pointer_labels,
           logits_labels, partial_pos, pointer_probs, log_probs,
           object_detections, object_features, object_boxes,
           src_table, tgt_table):
    batch, lq = queries.shape
    lv = values.shape[1]
    hidden = tgt_table.shape[1]

    q_flat = queries.reshape(-1).astype(jnp.int32)
    v_flat = values.reshape(-1).astype(jnp.int32)

    sc_q = _make_sc_gather_pack(batch * lq, hidden, 800)
    sc_v = _make_sc_gather_pack(batch * lv, hidden, 800)
    eq = sc_q(q_flat, tgt_table)
    ev = sc_v(v_flat, src_table)

    peq = _pos_encoding(lq, hidden)
    pev = _pos_encoding(lv, hidden)

    b = _make_tc_b(batch, lq, hidden, 256)(absolute_positions, eq, peq)
    d = _make_tc_d(batch, lv, hidden, 256)(ev, pev)

    return (b, d, queries_mask, values_mask, ids, permutation,
            absolute_positions, relative_positions, pointer_labels,
            logits_labels, partial_pos, pointer_probs, log_probs,
            object_detections, object_features, object_boxes)
